# baseline (speedup 1.0000x reference)
import jax
import jax.numpy as jnp
from jax import lax
from jax.experimental import pallas as pl
from jax.experimental.pallas import tpu as pltpu
from jax.experimental.pallas import tpu_sc as plsc

B, L, D = 4096, 200, 64
TOK = B * L
NC, NS = 2, 16
NW = NC * NS
PER_W = TOK // NW
CHUNK = 128
NCHUNK = PER_W // CHUNK

_mesh = plsc.VectorSubcoreMesh(
    core_axis_name="c", subcore_axis_name="s", num_cores=NC, num_subcores=NS
)


def _gather_body(idx_hbm, w_hbm, out_hbm, idx_v, rows_v, sem):
    wid = lax.axis_index("s") * NC + lax.axis_index("c")
    base = wid * PER_W

    def body(c, carry):
        off = base + c * CHUNK
        pltpu.sync_copy(idx_hbm.at[pl.ds(off, CHUNK)], idx_v)
        pltpu.async_copy(w_hbm.at[idx_v], rows_v, sem).wait()
        pltpu.sync_copy(rows_v, out_hbm.at[pl.ds(off, CHUNK)])
        return carry

    lax.fori_loop(0, NCHUNK, body, 0)


_gather = pl.kernel(
    _gather_body,
    out_type=jax.ShapeDtypeStruct((TOK, D), jnp.float32),
    mesh=_mesh,
    scratch_types=[
        pltpu.VMEM((CHUNK,), jnp.int32),
        pltpu.VMEM((CHUNK, D), jnp.float32),
        pltpu.SemaphoreType.DMA,
    ],
    compiler_params=pltpu.CompilerParams(use_tc_tiling_on_sc=False),
)

TB = 8


def _tr_body(emb_ref, out_ref):
    out_ref[...] = jnp.transpose(emb_ref[...], (0, 2, 1))


_transpose = pl.pallas_call(
    _tr_body,
    out_shape=jax.ShapeDtypeStruct((B, D, L), jnp.float32),
    grid=(B // TB,),
    in_specs=[pl.BlockSpec((TB, L, D), lambda i: (i, 0, 0))],
    out_specs=pl.BlockSpec((TB, D, L), lambda i: (i, 0, 0)),
)


def kernel(x, weight):
    idx = x.reshape(TOK).astype(jnp.int32)
    emb = _gather(idx, weight)
    return _transpose(emb.reshape(B, L, D))


# speedup vs baseline: 1.2296x; 1.2296x over previous
import jax
import jax.numpy as jnp
from jax import lax
from jax.experimental import pallas as pl
from jax.experimental.pallas import tpu as pltpu
from jax.experimental.pallas import tpu_sc as plsc

B, L, D = 4096, 200, 64
TOK = B * L
NC, NS = 2, 16
NW = NC * NS
BPW = B // NW
OUTB = D * L
G0, G1 = 128, L - 128

_mesh = plsc.VectorSubcoreMesh(
    core_axis_name="c", subcore_axis_name="s", num_cores=NC, num_subcores=NS
)


def _body(idx_hbm, w_hbm, out_hbm, idx0, idx1, rows0, rows1, t0, t1,
          sem_g0, sem_g1, sem_o0, sem_o1):
    wid = lax.axis_index("s") * NC + lax.axis_index("c")
    base = wid * BPW

    def load_idx(b, idxbuf):
        pltpu.sync_copy(idx_hbm.at[pl.ds(b * L, L)], idxbuf)

    def start_gather(idxbuf, rows, sem):
        pltpu.async_copy(w_hbm.at[idxbuf.at[pl.ds(0, G0)]],
                         rows.at[pl.ds(0, G0)], sem)
        pltpu.async_copy(w_hbm.at[idxbuf.at[pl.ds(G0, G1)]],
                         rows.at[pl.ds(G0, G1)], sem)

    def wait_gather(rows, sem):
        pltpu.make_async_copy(w_hbm.at[pl.ds(0, G0)],
                              rows.at[pl.ds(0, G0)], sem).wait()
        pltpu.make_async_copy(w_hbm.at[pl.ds(0, G1)],
                              rows.at[pl.ds(G0, G1)], sem).wait()

    def start_out(tbuf, b, sem):
        pltpu.async_copy(tbuf, out_hbm.at[pl.ds(b * OUTB, OUTB)], sem)

    def wait_out(tbuf, sem):
        pltpu.make_async_copy(out_hbm.at[pl.ds(0, OUTB)], tbuf, sem).wait()

    iota = lax.iota(jnp.int32, 16)
    flat_g = [(g * 16 + iota) * L for g in range(4)]

    def transpose_batch(rows, tbuf):
        def lbody(li, _):
            for u in range(4):
                l = 4 * li + u
                lb = jnp.broadcast_to(l, (16,))
                for g in range(4):
                    v = rows[l, pl.ds(g * 16, 16)]
                    plsc.store_scatter(tbuf, [flat_g[g] + lb], v)
            return 0
        lax.fori_loop(0, L // 4, lbody, 0)

    load_idx(base, idx0)
    start_gather(idx0, rows0, sem_g0)

    def body(i, _):
        bA = base + 2 * i
        bB = bA + 1
        load_idx(bB, idx1)
        start_gather(idx1, rows1, sem_g1)

        wait_gather(rows0, sem_g0)

        @pl.when(i > 0)
        def _():
            wait_out(t0, sem_o0)

        transpose_batch(rows0, t0)
        start_out(t0, bA, sem_o0)

        @pl.when(i < BPW // 2 - 1)
        def _():
            load_idx(bA + 2, idx0)
            start_gather(idx0, rows0, sem_g0)

        wait_gather(rows1, sem_g1)

        @pl.when(i > 0)
        def _():
            wait_out(t1, sem_o1)

        transpose_batch(rows1, t1)
        start_out(t1, bB, sem_o1)
        return 0

    lax.fori_loop(0, BPW // 2, body, 0)
    wait_out(t0, sem_o0)
    wait_out(t1, sem_o1)


_fused = pl.kernel(
    _body,
    out_type=jax.ShapeDtypeStruct((B * OUTB,), jnp.float32),
    mesh=_mesh,
    scratch_types=[
        pltpu.VMEM((L,), jnp.int32),
        pltpu.VMEM((L,), jnp.int32),
        pltpu.VMEM((L, D), jnp.float32),
        pltpu.VMEM((L, D), jnp.float32),
        pltpu.VMEM((OUTB,), jnp.float32),
        pltpu.VMEM((OUTB,), jnp.float32),
        pltpu.SemaphoreType.DMA,
        pltpu.SemaphoreType.DMA,
        pltpu.SemaphoreType.DMA,
        pltpu.SemaphoreType.DMA,
    ],
    compiler_params=pltpu.CompilerParams(
        use_tc_tiling_on_sc=False, needs_layout_passes=False
    ),
)


def kernel(x, weight):
    idx = x.reshape(TOK).astype(jnp.int32)
    out = _fused(idx, weight)
    return out.reshape(B, D, L)


# speedup vs baseline: 1.2448x; 1.0124x over previous
import jax
import jax.numpy as jnp
from jax import lax
from jax.experimental import pallas as pl
from jax.experimental.pallas import tpu as pltpu
from jax.experimental.pallas import tpu_sc as plsc

B, L, D = 4096, 200, 64
NV = 1_000_000
NC, NS = 2, 16
NW = NC * NS
LT = L // 8
CH = 256
NCHUNK = LT * 4
HALF = NCHUNK // 2

_mesh = plsc.VectorSubcoreMesh(
    core_axis_name="c", subcore_axis_name="s", num_cores=NC, num_subcores=NS
)


def _body(xt_hbm, w_hbm, out_hbm, xb0, xb1, rows0, rows1, t0, t1,
          sem_g0, sem_g1, sem_o0, sem_o1):
    wid = lax.axis_index("s") * NC + lax.axis_index("c")
    bcol = wid * 128

    def load_idx(c, xbuf):
        lrow = (c // 4) * 8 + (c % 4) * 2
        pltpu.sync_copy(xt_hbm.at[pl.ds(lrow, 2), pl.ds(bcol, 128)], xbuf)

    def start_gather(xbuf, rows, sem):
        pltpu.async_copy(w_hbm.at[xbuf.at[0]], rows.at[pl.ds(0, 128)], sem)
        pltpu.async_copy(w_hbm.at[xbuf.at[1]], rows.at[pl.ds(128, 128)], sem)

    def wait_gather(rows, sem):
        pltpu.make_async_copy(w_hbm.at[pl.ds(0, 128)],
                              rows.at[pl.ds(0, 128)], sem).wait()
        pltpu.make_async_copy(w_hbm.at[pl.ds(0, 128)],
                              rows.at[pl.ds(128, 128)], sem).wait()

    def start_out(tbuf, c, sem):
        lt = c // 4
        h = c % 4
        pltpu.async_copy(
            tbuf, out_hbm.at[:, lt, wid, pl.ds(h * CH, CH)], sem)

    def wait_out(tbuf, sem):
        pltpu.make_async_copy(out_hbm.at[:, 0, 0, pl.ds(0, CH)],
                              tbuf, sem).wait()

    iota = lax.iota(jnp.int32, 16)
    dgs = [g * 16 + iota for g in range(4)]

    def transpose_chunk(rows, tbuf):
        def ibody(ii, _):
            for u in range(2):
                i = 2 * ii + u
                ib = jnp.broadcast_to(i, (16,))
                for g in range(4):
                    v = rows[i, pl.ds(g * 16, 16)]
                    plsc.store_scatter(tbuf, [dgs[g], ib], v)
            return 0
        lax.fori_loop(0, CH // 2, ibody, 0)

    load_idx(0, xb0)
    start_gather(xb0, rows0, sem_g0)

    def body(i, _):
        cA = 2 * i
        cB = cA + 1
        load_idx(cB, xb1)
        start_gather(xb1, rows1, sem_g1)

        wait_gather(rows0, sem_g0)

        @pl.when(i > 0)
        def _():
            wait_out(t0, sem_o0)

        transpose_chunk(rows0, t0)
        start_out(t0, cA, sem_o0)

        @pl.when(i < HALF - 1)
        def _():
            load_idx(cA + 2, xb0)
            start_gather(xb0, rows0, sem_g0)

        wait_gather(rows1, sem_g1)

        @pl.when(i > 0)
        def _():
            wait_out(t1, sem_o1)

        transpose_chunk(rows1, t1)
        start_out(t1, cB, sem_o1)
        return 0

    lax.fori_loop(0, HALF, body, 0)
    wait_out(t0, sem_o0)
    wait_out(t1, sem_o1)


_fused = pl.kernel(
    _body,
    out_type=jax.ShapeDtypeStruct((D, LT, NW, 1024), jnp.float32),
    mesh=_mesh,
    scratch_types=[
        pltpu.VMEM((2, 128), jnp.int32),
        pltpu.VMEM((2, 128), jnp.int32),
        pltpu.VMEM((CH, D), jnp.float32),
        pltpu.VMEM((CH, D), jnp.float32),
        pltpu.VMEM((D, CH), jnp.float32),
        pltpu.VMEM((D, CH), jnp.float32),
        pltpu.SemaphoreType.DMA,
        pltpu.SemaphoreType.DMA,
        pltpu.SemaphoreType.DMA,
        pltpu.SemaphoreType.DMA,
    ],
    compiler_params=pltpu.CompilerParams(
        use_tc_tiling_on_sc=False, needs_layout_passes=False
    ),
)


def kernel(x, weight):
    xt = jnp.transpose(x * 2, (1, 0)).astype(jnp.int32)
    w2 = jnp.pad(weight, ((0, 0), (0, 64))).reshape(2 * NV, D)
    out = _fused(xt, w2)
    out = out.reshape(D, LT, NW, 8, 128)
    out = out.transpose(2, 4, 0, 1, 3)
    return out.reshape(B, D, L)
